# baseline (speedup 1.0000x reference)
import jax
import jax.numpy as jnp
from jax import lax
from jax.experimental import pallas as pl
from jax.experimental.pallas import tpu as pltpu

_R = 512


def _build_table(frame_table, group_idx, lit_positions):
    rot = jnp.transpose(frame_table[:, :, :3, :3], (2, 3, 1, 0)).reshape(72, 21)
    dt = jnp.transpose(frame_table[:, :, :3, 3], (2, 1, 0)).reshape(24, 21)
    lit = jnp.transpose(lit_positions, (2, 1, 0))
    pad3 = jnp.zeros((3, 21), jnp.float32)
    ridx = group_idx.T.astype(jnp.float32)
    T = jnp.concatenate([
        rot, dt,
        lit[0], pad3, lit[1], pad3, lit[2], pad3,
        ridx, pad3,
    ], axis=0)
    T = jnp.pad(T, ((0, 0), (0, 11)))
    hi = T.astype(jnp.bfloat16)
    lo = (T - hi.astype(jnp.float32)).astype(jnp.bfloat16)
    return jnp.concatenate([hi, lo], axis=0)


def _body(a_ref, s_ref, c_ref, t_ref, p_ref, o_ref):
    R = a_ref.shape[-1]
    f32 = jnp.float32
    aat = a_ref[0]
    H = (lax.broadcasted_iota(jnp.int32, (32, R), 0) == aat).astype(jnp.bfloat16)
    G2 = lax.dot_general(t_ref[...], H, (((1,), (0,)), ((), ())),
                         preferred_element_type=f32)
    G = G2[:256] + G2[256:]

    s8 = jnp.concatenate([jnp.zeros((1, R), f32), s_ref[...]], axis=0)
    c8 = jnp.concatenate([jnp.ones((1, R), f32), c_ref[...]], axis=0)

    D = [[G[(j * 3 + i) * 8:(j * 3 + i) * 8 + 8] for i in range(3)]
         for j in range(3)]
    T0 = [G[72 + i * 8:72 + i * 8 + 8] for i in range(3)]

    Rj = [[None] * 3 for _ in range(3)]
    for j in range(3):
        Rj[j][0] = D[j][0]
        Rj[j][1] = c8 * D[j][1] + s8 * D[j][2]
        Rj[j][2] = c8 * D[j][2] - s8 * D[j][1]

    r = [[[Rj[j][i][g:g + 1] for i in range(3)] for j in range(3)]
         for g in range(8)]
    t = [[T0[i][g:g + 1] for i in range(3)] for g in range(8)]

    for g in (5, 6, 7):
        p, q, tq, tp = r[g - 1], r[g], t[g], t[g - 1]
        r[g] = [[p[j][0] * q[0][i] + p[j][1] * q[1][i] + p[j][2] * q[2][i]
                 for i in range(3)] for j in range(3)]
        t[g] = [p[i][0] * tq[0] + p[i][1] * tq[1] + p[i][2] * tq[2] + tp[i]
                for i in range(3)]

    lit = [G[96 + 40 * j:96 + 40 * j + 37] for j in range(3)]
    ridx = G[216:253]

    acc = [jnp.zeros((37, R), f32) for _ in range(3)]
    for g in range(8):
        mf = (ridx == f32(g)).astype(f32)
        ml = [mf * lit[j] for j in range(3)]
        for i in range(3):
            acc[i] = (acc[i] + ml[0] * r[g][0][i] + ml[1] * r[g][1][i]
                      + ml[2] * r[g][2][i] + mf * t[g][i])

    Z = None
    for i in range(3):
        ah = acc[i].astype(jnp.bfloat16)
        al = (acc[i] - ah.astype(f32)).astype(jnp.bfloat16)
        pt = p_ref[i]
        for m in (ah, al):
            d = lax.dot_general(m, pt, (((0,), (0,)), ((), ())),
                                preferred_element_type=f32)
            Z = d if Z is None else Z + d
    o_ref[...] = Z


def _run(aat3, sin_t, cos_t, table, pts, interpret=False):
    NB = aat3.shape[0]
    R = aat3.shape[-1]
    N = NB * R
    return pl.pallas_call(
        _body,
        grid=(NB,),
        in_specs=[
            pl.BlockSpec((1, 1, R), lambda i: (i, 0, 0)),
            pl.BlockSpec((7, R), lambda i: (0, i)),
            pl.BlockSpec((7, R), lambda i: (0, i)),
            pl.BlockSpec((512, 32), lambda i: (0, 0)),
            pl.BlockSpec((3, 37, 111), lambda i: (0, 0, 0)),
        ],
        out_specs=pl.BlockSpec((R, 111), lambda i: (i, 0)),
        out_shape=jax.ShapeDtypeStruct((N, 111), jnp.float32),
        compiler_params=pltpu.CompilerParams(
            dimension_semantics=("parallel",)),
        interpret=interpret,
    )(aat3, sin_t, cos_t, table, pts)


def kernel(aatype, sin_cos, frame_table, group_idx, lit_positions):
    B, L = aatype.shape
    N = B * L
    R = _R
    NB = N // R
    aat3 = aatype.astype(jnp.int32).reshape(NB, 1, R)
    sc = sin_cos.reshape(N, 7, 2)
    sin_t = jnp.transpose(sc[:, :, 0])
    cos_t = jnp.transpose(sc[:, :, 1])
    table = _build_table(frame_table.astype(jnp.float32), group_idx,
                         lit_positions.astype(jnp.float32))
    a_iota = jnp.arange(37)[:, None]
    c_iota = jnp.arange(111)[None, :]
    pts = jnp.stack([(c_iota == a_iota * 3 + i).astype(jnp.bfloat16)
                     for i in range(3)], axis=0)
    out = _run(aat3, sin_t, cos_t, table, pts)
    return out.reshape(B, L, 37, 3)


# speedup vs baseline: 1.2020x; 1.2020x over previous
import jax
import jax.numpy as jnp
from jax import lax
from jax.experimental import pallas as pl
from jax.experimental.pallas import tpu as pltpu

_R = 1024


def _build_table(frame_table, group_idx, lit_positions):
    rot = jnp.transpose(frame_table[:, :, :3, :3], (2, 3, 1, 0)).reshape(72, 21)
    dt = jnp.transpose(frame_table[:, :, :3, 3], (2, 1, 0)).reshape(24, 21)
    lit = jnp.transpose(lit_positions, (2, 1, 0))
    pad3 = jnp.zeros((3, 21), jnp.float32)
    ridx = group_idx.T.astype(jnp.float32)
    T = jnp.concatenate([
        rot, dt,
        lit[0], pad3, lit[1], pad3, lit[2], pad3,
        ridx, pad3,
    ], axis=0)
    T = jnp.pad(T, ((0, 0), (0, 11)))
    hi = T.astype(jnp.bfloat16)
    lo = (T - hi.astype(jnp.float32)).astype(jnp.bfloat16)
    return jnp.concatenate([hi, lo], axis=0)


def _body(a_ref, sc_ref, t_ref, o_ref):
    R = a_ref.shape[-1]
    f32 = jnp.float32
    aat = a_ref[0]
    H = (lax.broadcasted_iota(jnp.int32, (32, R), 0) == aat).astype(jnp.bfloat16)
    G2 = lax.dot_general(t_ref[...], H, (((1,), (0,)), ((), ())),
                         preferred_element_type=f32)
    G = G2[:256] + G2[256:]

    s8 = jnp.concatenate([jnp.zeros((1, R), f32), sc_ref[0]], axis=0)
    c8 = jnp.concatenate([jnp.ones((1, R), f32), sc_ref[1]], axis=0)

    D = [[G[(j * 3 + i) * 8:(j * 3 + i) * 8 + 8] for i in range(3)]
         for j in range(3)]
    T0 = [G[72 + i * 8:72 + i * 8 + 8] for i in range(3)]

    Rj = [[None] * 3 for _ in range(3)]
    for j in range(3):
        Rj[j][0] = D[j][0]
        Rj[j][1] = c8 * D[j][1] + s8 * D[j][2]
        Rj[j][2] = c8 * D[j][2] - s8 * D[j][1]

    r = [[[Rj[j][i][g:g + 1] for i in range(3)] for j in range(3)]
         for g in range(8)]
    t = [[T0[i][g:g + 1] for i in range(3)] for g in range(8)]

    for g in (5, 6, 7):
        p, q, tq, tp = r[g - 1], r[g], t[g], t[g - 1]
        r[g] = [[p[j][0] * q[0][i] + p[j][1] * q[1][i] + p[j][2] * q[2][i]
                 for i in range(3)] for j in range(3)]
        t[g] = [p[i][0] * tq[0] + p[i][1] * tq[1] + p[i][2] * tq[2] + tp[i]
                for i in range(3)]

    lit = [G[96 + 40 * j:96 + 40 * j + 37] for j in range(3)]
    ridx = G[216:253]

    acc = [jnp.zeros((37, R), f32) for _ in range(3)]
    for g in range(8):
        mf = (ridx == f32(g)).astype(f32)
        ml = [mf * lit[j] for j in range(3)]
        for i in range(3):
            acc[i] = (acc[i] + ml[0] * r[g][0][i] + ml[1] * r[g][1][i]
                      + ml[2] * r[g][2][i] + mf * t[g][i])

    for i in range(3):
        o_ref[i] = acc[i]


def _run(aat3, sc_t, table, interpret=False):
    NB = aat3.shape[0]
    R = aat3.shape[-1]
    N = NB * R
    return pl.pallas_call(
        _body,
        grid=(NB,),
        in_specs=[
            pl.BlockSpec((1, 1, R), lambda i: (i, 0, 0)),
            pl.BlockSpec((2, 7, R), lambda i: (0, 0, i)),
            pl.BlockSpec((512, 32), lambda i: (0, 0)),
        ],
        out_specs=pl.BlockSpec((3, 37, R), lambda i: (0, 0, i)),
        out_shape=jax.ShapeDtypeStruct((3, 37, N), jnp.float32),
        compiler_params=pltpu.CompilerParams(
            dimension_semantics=("parallel",)),
        interpret=interpret,
    )(aat3, sc_t, table)


def kernel(aatype, sin_cos, frame_table, group_idx, lit_positions):
    B, L = aatype.shape
    N = B * L
    R = _R
    NB = N // R
    aat3 = aatype.astype(jnp.int32).reshape(NB, 1, R)
    sc_t = jnp.transpose(sin_cos.reshape(N, 7, 2), (2, 1, 0))
    table = _build_table(frame_table.astype(jnp.float32), group_idx,
                         lit_positions.astype(jnp.float32))
    out = _run(aat3, sc_t, table)
    return jnp.transpose(out, (2, 1, 0)).reshape(B, L, 37, 3)


# speedup vs baseline: 1.2335x; 1.0262x over previous
import jax
import jax.numpy as jnp
from jax import lax
from jax.experimental import pallas as pl
from jax.experimental.pallas import tpu as pltpu

_R = 2048


def _build_table(frame_table, group_idx, lit_positions):
    rot = jnp.transpose(frame_table[:, :, :3, :3], (2, 3, 1, 0)).reshape(72, 21)
    dt = jnp.transpose(frame_table[:, :, :3, 3], (2, 1, 0)).reshape(24, 21)
    lit = jnp.transpose(lit_positions, (2, 1, 0))
    pad3 = jnp.zeros((3, 21), jnp.float32)
    ridx = group_idx.T.astype(jnp.float32)
    T = jnp.concatenate([
        rot, dt,
        lit[0], pad3, lit[1], pad3, lit[2], pad3,
        ridx, pad3,
    ], axis=0)
    T = jnp.pad(T, ((0, 0), (0, 11)))
    hi = T.astype(jnp.bfloat16)
    lo = (T - hi.astype(jnp.float32)).astype(jnp.bfloat16)
    return jnp.concatenate([hi, lo], axis=0)


def _body(a_ref, sc_ref, t_ref, o_ref):
    R = a_ref.shape[-1]
    f32 = jnp.float32
    aat = a_ref[0]
    H = (lax.broadcasted_iota(jnp.int32, (32, R), 0) == aat).astype(jnp.bfloat16)
    G2 = lax.dot_general(t_ref[...], H, (((1,), (0,)), ((), ())),
                         preferred_element_type=f32)
    G = G2[:256] + G2[256:]

    s8 = jnp.concatenate([jnp.zeros((1, R), f32), sc_ref[0]], axis=0)
    c8 = jnp.concatenate([jnp.ones((1, R), f32), sc_ref[1]], axis=0)

    D = [[G[(j * 3 + i) * 8:(j * 3 + i) * 8 + 8] for i in range(3)]
         for j in range(3)]
    T0 = [G[72 + i * 8:72 + i * 8 + 8] for i in range(3)]

    Rj = [[None] * 3 for _ in range(3)]
    for j in range(3):
        Rj[j][0] = D[j][0]
        Rj[j][1] = c8 * D[j][1] + s8 * D[j][2]
        Rj[j][2] = c8 * D[j][2] - s8 * D[j][1]

    r = [[[Rj[j][i][g:g + 1] for i in range(3)] for j in range(3)]
         for g in range(8)]
    t = [[T0[i][g:g + 1] for i in range(3)] for g in range(8)]

    for g in (5, 6, 7):
        p, q, tq, tp = r[g - 1], r[g], t[g], t[g - 1]
        r[g] = [[p[j][0] * q[0][i] + p[j][1] * q[1][i] + p[j][2] * q[2][i]
                 for i in range(3)] for j in range(3)]
        t[g] = [p[i][0] * tq[0] + p[i][1] * tq[1] + p[i][2] * tq[2] + tp[i]
                for i in range(3)]

    lit = [G[96 + 40 * j:96 + 40 * j + 37] for j in range(3)]
    ridx = G[216:253]

    acc = [jnp.zeros((37, R), f32) for _ in range(3)]
    for g in range(8):
        mf = (ridx == f32(g)).astype(f32)
        ml = [mf * lit[j] for j in range(3)]
        for i in range(3):
            acc[i] = (acc[i] + ml[0] * r[g][0][i] + ml[1] * r[g][1][i]
                      + ml[2] * r[g][2][i] + mf * t[g][i])

    for i in range(3):
        o_ref[i] = acc[i]


def _run(aat3, sc_t, table, interpret=False):
    NB = aat3.shape[0]
    R = aat3.shape[-1]
    N = NB * R
    return pl.pallas_call(
        _body,
        grid=(NB,),
        in_specs=[
            pl.BlockSpec((1, 1, R), lambda i: (i, 0, 0)),
            pl.BlockSpec((2, 7, R), lambda i: (0, 0, i)),
            pl.BlockSpec((512, 32), lambda i: (0, 0)),
        ],
        out_specs=pl.BlockSpec((3, 37, R), lambda i: (0, 0, i)),
        out_shape=jax.ShapeDtypeStruct((3, 37, N), jnp.float32),
        compiler_params=pltpu.CompilerParams(
            dimension_semantics=("parallel",)),
        interpret=interpret,
    )(aat3, sc_t, table)


def kernel(aatype, sin_cos, frame_table, group_idx, lit_positions):
    B, L = aatype.shape
    N = B * L
    R = _R
    NB = N // R
    aat3 = aatype.astype(jnp.int32).reshape(NB, 1, R)
    sc_t = jnp.transpose(sin_cos.reshape(N, 7, 2), (2, 1, 0))
    table = _build_table(frame_table.astype(jnp.float32), group_idx,
                         lit_positions.astype(jnp.float32))
    out = _run(aat3, sc_t, table)
    return jnp.transpose(out, (2, 1, 0)).reshape(B, L, 37, 3)


# speedup vs baseline: 1.4213x; 1.1522x over previous
import jax
import jax.numpy as jnp
from jax import lax
from jax.experimental import pallas as pl
from jax.experimental.pallas import tpu as pltpu

_R = 2048


def _build_table(frame_table, group_idx, lit_positions):
    rot = jnp.transpose(frame_table[:, :, :3, :3], (2, 3, 1, 0)).reshape(72, 21)
    dt = jnp.transpose(frame_table[:, :, :3, 3], (2, 1, 0)).reshape(24, 21)
    lit = jnp.transpose(lit_positions, (2, 1, 0))
    pad3 = jnp.zeros((3, 21), jnp.float32)
    ridx = group_idx.T.astype(jnp.float32)
    T = jnp.concatenate([
        rot, dt,
        lit[0], pad3, lit[1], pad3, lit[2], pad3,
        ridx, pad3,
    ], axis=0)
    T = jnp.pad(T, ((0, 0), (0, 11)))
    hi = T.astype(jnp.bfloat16)
    lo = (T - hi.astype(jnp.float32)).astype(jnp.bfloat16)
    return jnp.concatenate([hi, lo], axis=0)


def _body(a_ref, sc_ref, t_ref, o_ref):
    R = a_ref.shape[-1]
    f32 = jnp.float32
    aat = a_ref[0]
    H = (lax.broadcasted_iota(jnp.int32, (32, R), 0) == aat).astype(jnp.bfloat16)
    G2 = lax.dot_general(t_ref[...], H, (((1,), (0,)), ((), ())),
                         preferred_element_type=f32)
    G = G2[:256] + G2[256:]

    s8 = jnp.concatenate([jnp.zeros((1, R), f32), sc_ref[0]], axis=0)
    c8 = jnp.concatenate([jnp.ones((1, R), f32), sc_ref[1]], axis=0)

    D = [[G[(j * 3 + i) * 8:(j * 3 + i) * 8 + 8] for i in range(3)]
         for j in range(3)]
    T0 = [G[72 + i * 8:72 + i * 8 + 8] for i in range(3)]

    Rj = [[None] * 3 for _ in range(3)]
    for j in range(3):
        Rj[j][0] = D[j][0]
        Rj[j][1] = c8 * D[j][1] + s8 * D[j][2]
        Rj[j][2] = c8 * D[j][2] - s8 * D[j][1]

    r = [[[Rj[j][i][g:g + 1] for i in range(3)] for j in range(3)]
         for g in range(8)]
    t = [[T0[i][g:g + 1] for i in range(3)] for g in range(8)]

    for g in (5, 6, 7):
        p, q, tq, tp = r[g - 1], r[g], t[g], t[g - 1]
        r[g] = [[p[j][0] * q[0][i] + p[j][1] * q[1][i] + p[j][2] * q[2][i]
                 for i in range(3)] for j in range(3)]
        t[g] = [p[i][0] * tq[0] + p[i][1] * tq[1] + p[i][2] * tq[2] + tp[i]
                for i in range(3)]

    lit = [G[96 + 40 * j:96 + 40 * j + 37] for j in range(3)]
    ridx = G[216:253]

    b2 = ridx >= f32(4)
    rem = ridx - jnp.where(b2, f32(4), f32(0))
    b1 = rem >= f32(2)
    rem2 = rem - jnp.where(b1, f32(2), f32(0))
    b0 = rem2 >= f32(1)

    def sel3(vals):
        l1 = [jnp.where(b0, vals[2 * k + 1], vals[2 * k]) for k in range(4)]
        l2 = [jnp.where(b1, l1[2 * k + 1], l1[2 * k]) for k in range(2)]
        return jnp.where(b2, l2[1], l2[0])

    Rs = [[sel3([r[g][j][i] for g in range(8)]) for i in range(3)]
          for j in range(3)]
    Ts = [sel3([t[g][i] for g in range(8)]) for i in range(3)]
    for i in range(3):
        o_ref[i] = (lit[0] * Rs[0][i] + lit[1] * Rs[1][i]
                    + lit[2] * Rs[2][i] + Ts[i])


def _run(aat3, sc_t, table, interpret=False):
    NB = aat3.shape[0]
    R = aat3.shape[-1]
    N = NB * R
    return pl.pallas_call(
        _body,
        grid=(NB,),
        in_specs=[
            pl.BlockSpec((1, 1, R), lambda i: (i, 0, 0)),
            pl.BlockSpec((2, 7, R), lambda i: (0, 0, i)),
            pl.BlockSpec((512, 32), lambda i: (0, 0)),
        ],
        out_specs=pl.BlockSpec((3, 37, R), lambda i: (0, 0, i)),
        out_shape=jax.ShapeDtypeStruct((3, 37, N), jnp.float32),
        compiler_params=pltpu.CompilerParams(
            dimension_semantics=("parallel",)),
        interpret=interpret,
    )(aat3, sc_t, table)


def kernel(aatype, sin_cos, frame_table, group_idx, lit_positions):
    B, L = aatype.shape
    N = B * L
    R = _R
    NB = N // R
    aat3 = aatype.astype(jnp.int32).reshape(NB, 1, R)
    sc_t = jnp.transpose(sin_cos.reshape(N, 7, 2), (2, 1, 0))
    table = _build_table(frame_table.astype(jnp.float32), group_idx,
                         lit_positions.astype(jnp.float32))
    out = _run(aat3, sc_t, table)
    return jnp.transpose(out, (2, 1, 0)).reshape(B, L, 37, 3)


# speedup vs baseline: 1.6142x; 1.1358x over previous
import jax
import jax.numpy as jnp
from jax import lax
from jax.experimental import pallas as pl
from jax.experimental.pallas import tpu as pltpu

_R = 2048


def _build_table(frame_table, group_idx, lit_positions):
    rot = jnp.transpose(frame_table[:, :, :3, :3], (2, 3, 1, 0)).reshape(72, 21)
    dt = jnp.transpose(frame_table[:, :, :3, 3], (2, 1, 0)).reshape(24, 21)
    lit = jnp.transpose(lit_positions, (2, 1, 0))
    pad3 = jnp.zeros((3, 21), jnp.float32)
    ridx = group_idx.T.astype(jnp.float32)
    T = jnp.concatenate([
        rot, dt,
        lit[0], pad3, lit[1], pad3, lit[2], pad3,
        ridx, pad3,
    ], axis=0)
    T = jnp.pad(T, ((0, 0), (0, 11)))
    hi = T.astype(jnp.bfloat16)
    lo = (T - hi.astype(jnp.float32)).astype(jnp.bfloat16)
    return jnp.concatenate([hi, lo], axis=0)


def _body(a_ref, sc_ref, t_ref, p_ref, o_ref):
    R = a_ref.shape[-1]
    f32 = jnp.float32
    aat = a_ref[0]
    H = (lax.broadcasted_iota(jnp.int32, (32, R), 0) == aat).astype(jnp.bfloat16)
    G2 = lax.dot_general(t_ref[...], H, (((1,), (0,)), ((), ())),
                         preferred_element_type=f32)
    G = G2[:256] + G2[256:]

    s8 = jnp.concatenate([jnp.zeros((1, R), f32), sc_ref[0]], axis=0)
    c8 = jnp.concatenate([jnp.ones((1, R), f32), sc_ref[1]], axis=0)

    D = [[G[(j * 3 + i) * 8:(j * 3 + i) * 8 + 8] for i in range(3)]
         for j in range(3)]
    T0 = [G[72 + i * 8:72 + i * 8 + 8] for i in range(3)]

    Rj = [[None] * 3 for _ in range(3)]
    for j in range(3):
        Rj[j][0] = D[j][0]
        Rj[j][1] = c8 * D[j][1] + s8 * D[j][2]
        Rj[j][2] = c8 * D[j][2] - s8 * D[j][1]

    r = [[[Rj[j][i][g:g + 1] for i in range(3)] for j in range(3)]
         for g in range(8)]
    t = [[T0[i][g:g + 1] for i in range(3)] for g in range(8)]

    for g in (5, 6, 7):
        p, q, tq, tp = r[g - 1], r[g], t[g], t[g - 1]
        r[g] = [[p[j][0] * q[0][i] + p[j][1] * q[1][i] + p[j][2] * q[2][i]
                 for i in range(3)] for j in range(3)]
        t[g] = [p[i][0] * tq[0] + p[i][1] * tq[1] + p[i][2] * tq[2] + tp[i]
                for i in range(3)]

    lit = [G[96 + 40 * j:96 + 40 * j + 37] for j in range(3)]
    ridx = G[216:253]

    b2 = ridx >= f32(4)
    rem = ridx - jnp.where(b2, f32(4), f32(0))
    b1 = rem >= f32(2)
    rem2 = rem - jnp.where(b1, f32(2), f32(0))
    b0 = rem2 >= f32(1)

    def sel3(vals):
        l1 = [jnp.where(b0, vals[2 * k + 1], vals[2 * k]) for k in range(4)]
        l2 = [jnp.where(b1, l1[2 * k + 1], l1[2 * k]) for k in range(2)]
        return jnp.where(b2, l2[1], l2[0])

    Rs = [[sel3([r[g][j][i] for g in range(8)]) for i in range(3)]
          for j in range(3)]
    Ts = [sel3([t[g][i] for g in range(8)]) for i in range(3)]

    bf16 = jnp.bfloat16
    zpad = jnp.zeros((3, R), bf16)
    parts = []
    for i in range(3):
        oi = (lit[0] * Rs[0][i] + lit[1] * Rs[1][i]
              + lit[2] * Rs[2][i] + Ts[i])
        oh = oi.astype(bf16)
        ol = (oi - oh.astype(f32)).astype(bf16)
        parts += [oh, zpad, ol, zpad]
    X = jnp.concatenate(parts, axis=0)
    o_ref[...] = lax.dot_general(X, p_ref[...], (((0,), (0,)), ((), ())),
                                 preferred_element_type=f32)


def _run(aat3, sc_t, table, pt, interpret=False):
    NB = aat3.shape[0]
    R = aat3.shape[-1]
    N = NB * R
    return pl.pallas_call(
        _body,
        grid=(NB,),
        in_specs=[
            pl.BlockSpec((1, 1, R), lambda i: (i, 0, 0)),
            pl.BlockSpec((2, 7, R), lambda i: (0, 0, i)),
            pl.BlockSpec((512, 32), lambda i: (0, 0)),
            pl.BlockSpec((240, 111), lambda i: (0, 0)),
        ],
        out_specs=pl.BlockSpec((R, 111), lambda i: (i, 0)),
        out_shape=jax.ShapeDtypeStruct((N, 111), jnp.float32),
        compiler_params=pltpu.CompilerParams(
            dimension_semantics=("parallel",)),
        interpret=interpret,
    )(aat3, sc_t, table, pt)


def kernel(aatype, sin_cos, frame_table, group_idx, lit_positions):
    B, L = aatype.shape
    N = B * L
    R = _R
    NB = N // R
    aat3 = aatype.astype(jnp.int32).reshape(NB, 1, R)
    sc_t = jnp.transpose(sin_cos.reshape(N, 7, 2), (2, 1, 0))
    table = _build_table(frame_table.astype(jnp.float32), group_idx,
                         lit_positions.astype(jnp.float32))
    rows = jnp.arange(240)
    a = rows % 40
    col = a * 3 + rows // 80
    pt = ((jnp.arange(111)[None, :] == col[:, None])
          & (a < 37)[:, None]).astype(jnp.bfloat16)
    out = _run(aat3, sc_t, table, pt)
    return out.reshape(B, L, 37, 3)
